# Initial kernel scaffold; baseline (speedup 1.0000x reference)
import jax
import jax.numpy as jnp
from jax.experimental import pallas as pl


def kernel(x, edge_index, W, b):
    raise NotImplementedError("write your pallas kernel here")


# speedup vs baseline: 11.5240x; 11.5240x over previous
import functools

import jax
import jax.numpy as jnp
from jax import lax
from jax.experimental import pallas as pl
from jax.experimental.pallas import tpu as pltpu
from jax.experimental.pallas import tpu_sc as plsc

N_NODES = 10000
N_EDGES = 320000
D = 128

NC = 2
NS = 16
NW = NC * NS
CHUNK = 128
CHUNKS_PER_W = 80
E_PAD = NW * CHUNK * CHUNKS_PER_W
N_DUMP = 16
N_ACC = 10240


def _deg_body(row2d, deg_out, deg_acc, rstage, ones_v, zeros_v):
    c = lax.axis_index("c")
    s = lax.axis_index("s")
    wid = c * NS + s
    for j in range(CHUNK // 16):
        ones_v[pl.ds(j * 16, 16)] = jnp.ones((16,), jnp.float32)
        zeros_v[pl.ds(j * 16, 16)] = jnp.zeros((16,), jnp.float32)
    for k in range(N_ACC // NS // CHUNK):
        pltpu.sync_copy(zeros_v, deg_acc.at[pl.ds(s * (N_ACC // NS) + k * CHUNK, CHUNK)])
    plsc.subcore_barrier()
    pltpu.sync_copy(row2d.at[pl.ds(wid * CHUNKS_PER_W, CHUNKS_PER_W)], rstage)

    def body(g, carry):
        pltpu.sync_copy(ones_v, deg_acc.at[rstage.at[g]], add=True)
        return carry

    lax.fori_loop(0, CHUNKS_PER_W, body, 0)
    plsc.subcore_barrier()
    pltpu.sync_copy(deg_acc.at[pl.ds(s * (N_ACC // NS), N_ACC // NS)],
                    deg_out.at[pl.ds(c * N_ACC + s * (N_ACC // NS), N_ACC // NS)])


def _deg_kernel(row2d):
    return pl.kernel(
        _deg_body,
        out_type=jax.ShapeDtypeStruct((NC * N_ACC,), jnp.float32),
        mesh=plsc.VectorSubcoreMesh(core_axis_name="c", subcore_axis_name="s"),
        scratch_types=[
            pltpu.VMEM_SHARED((N_ACC,), jnp.float32),
            pltpu.VMEM((CHUNKS_PER_W, CHUNK), jnp.int32),
            pltpu.VMEM((CHUNK,), jnp.float32),
            pltpu.VMEM((CHUNK,), jnp.float32),
        ],
    )(row2d)


def _edge_body(col2d, row2d, z, out_p, acc, cstage, rstage, rows_v, sem):
    c = lax.axis_index("c")
    s = lax.axis_index("s")
    wid = c * NS + s
    def zbody(i, carry):
        for j in range(D // 16):
            rows_v[i, pl.ds(j * 16, 16)] = jnp.zeros((16,), jnp.float32)
        return carry

    lax.fori_loop(0, CHUNK, zbody, 0)
    for k in range(N_ACC // NS // CHUNK):
        pltpu.sync_copy(rows_v, acc.at[pl.ds(s * (N_ACC // NS) + k * CHUNK, CHUNK)])
    plsc.subcore_barrier()
    pltpu.sync_copy(col2d.at[pl.ds(wid * CHUNKS_PER_W, CHUNKS_PER_W)], cstage)
    pltpu.sync_copy(row2d.at[pl.ds(wid * CHUNKS_PER_W, CHUNKS_PER_W)], rstage)

    def body(g, carry):
        pltpu.async_copy(z.at[cstage.at[g]], rows_v, sem).wait()
        pltpu.sync_copy(rows_v, acc.at[rstage.at[g]], add=True)
        return carry

    lax.fori_loop(0, CHUNKS_PER_W, body, 0)
    plsc.subcore_barrier()
    rows_per_tile = N_ACC // NS
    pltpu.sync_copy(acc.at[pl.ds(s * rows_per_tile, rows_per_tile)],
                    out_p.at[c, pl.ds(s * rows_per_tile, rows_per_tile)])


def _edge_kernel(col2d, row2d, z):
    return pl.kernel(
        _edge_body,
        out_type=jax.ShapeDtypeStruct((NC, N_ACC, D), jnp.float32),
        mesh=plsc.VectorSubcoreMesh(core_axis_name="c", subcore_axis_name="s"),
        scratch_types=[
            pltpu.VMEM_SHARED((N_ACC, D), jnp.float32),
            pltpu.VMEM((CHUNKS_PER_W, CHUNK), jnp.int32),
            pltpu.VMEM((CHUNKS_PER_W, CHUNK), jnp.int32),
            pltpu.VMEM((CHUNK, D), jnp.float32),
            pltpu.SemaphoreType.DMA,
        ],
    )(col2d, row2d, z)


def _dis_from(deg_ref):
    d = deg_ref[..., 0:1] + deg_ref[..., 1:2]
    return jnp.where(d > 0.0, lax.rsqrt(d), 0.0)


def _z_body(x_ref, w_ref, deg_ref, o_ref):
    mm = lax.dot_general(x_ref[...], w_ref[...], (((1,), (1,)), ((), ())),
                         preferred_element_type=jnp.float32)
    o_ref[...] = mm * _dis_from(deg_ref)


def _z_kernel(x, w, deg_t):
    blk = 1000
    grid = N_NODES // blk
    return pl.pallas_call(
        _z_body,
        grid=(grid,),
        in_specs=[
            pl.BlockSpec((blk, D), lambda i: (i, 0)),
            pl.BlockSpec((D, D), lambda i: (0, 0)),
            pl.BlockSpec((blk, NC), lambda i: (i, 0)),
        ],
        out_specs=pl.BlockSpec((blk, D), lambda i: (i, 0)),
        out_shape=jax.ShapeDtypeStruct((N_NODES, D), jnp.float32),
    )(x, w, deg_t)


def _final_body(p_ref, deg_ref, b_ref, o_ref):
    acc = p_ref[0] + p_ref[1]
    o_ref[...] = acc * _dis_from(deg_ref) + b_ref[...]


def _final_kernel(out_p, deg_t, b2):
    blk = 1000
    grid = N_NODES // blk
    return pl.pallas_call(
        _final_body,
        grid=(grid,),
        in_specs=[
            pl.BlockSpec((NC, blk, D), lambda i: (0, i, 0)),
            pl.BlockSpec((blk, NC), lambda i: (i, 0)),
            pl.BlockSpec((1, D), lambda i: (0, 0)),
        ],
        out_specs=pl.BlockSpec((blk, D), lambda i: (i, 0)),
        out_shape=jax.ShapeDtypeStruct((N_NODES, D), jnp.float32),
    )(out_p, deg_t, b2)


@jax.jit
def kernel(x, edge_index, W, b):
    row = edge_index[0]
    col = edge_index[1]
    pad = E_PAD - N_EDGES
    pad_row = N_NODES + (jnp.arange(pad, dtype=jnp.int32) % N_DUMP)
    pad_col = jnp.zeros((pad,), jnp.int32)
    row2d = jnp.concatenate([row, pad_row]).reshape(E_PAD // CHUNK, CHUNK)
    col2d = jnp.concatenate([col, pad_col]).reshape(E_PAD // CHUNK, CHUNK)

    deg_p = _deg_kernel(row2d).reshape(NC, N_ACC)
    deg_t = deg_p[:, :N_NODES].T
    z = _z_kernel(x, W, deg_t)
    out_p = _edge_kernel(col2d, row2d, z)
    return _final_kernel(out_p[:, :N_NODES], deg_t, b.reshape(1, D))


# speedup vs baseline: 12.0582x; 1.0464x over previous
import functools

import jax
import jax.numpy as jnp
from jax import lax
from jax.experimental import pallas as pl
from jax.experimental.pallas import tpu as pltpu
from jax.experimental.pallas import tpu_sc as plsc

N_NODES = 10000
N_EDGES = 320000
D = 128

NC = 2
NS = 16
NW = NC * NS
CHUNK = 128
CHUNKS_PER_W = 80
E_PAD = NW * CHUNK * CHUNKS_PER_W
N_DUMP = 16
N_ACC = 10240


def _deg_body(row2d, deg_out, deg_acc, rstage, ones_v, zeros_v):
    c = lax.axis_index("c")
    s = lax.axis_index("s")
    wid = c * NS + s
    for j in range(CHUNK // 16):
        ones_v[pl.ds(j * 16, 16)] = jnp.ones((16,), jnp.float32)
        zeros_v[pl.ds(j * 16, 16)] = jnp.zeros((16,), jnp.float32)
    for k in range(N_ACC // NS // CHUNK):
        pltpu.sync_copy(zeros_v, deg_acc.at[pl.ds(s * (N_ACC // NS) + k * CHUNK, CHUNK)])
    plsc.subcore_barrier()
    pltpu.sync_copy(row2d.at[pl.ds(wid * CHUNKS_PER_W, CHUNKS_PER_W)], rstage)

    def body(g, carry):
        pltpu.sync_copy(ones_v, deg_acc.at[rstage.at[g]], add=True)
        return carry

    lax.fori_loop(0, CHUNKS_PER_W, body, 0)
    plsc.subcore_barrier()
    pltpu.sync_copy(deg_acc.at[pl.ds(s * (N_ACC // NS), N_ACC // NS)],
                    deg_out.at[pl.ds(c * N_ACC + s * (N_ACC // NS), N_ACC // NS)])


def _deg_kernel(row2d):
    return pl.kernel(
        _deg_body,
        out_type=jax.ShapeDtypeStruct((NC * N_ACC,), jnp.float32),
        mesh=plsc.VectorSubcoreMesh(core_axis_name="c", subcore_axis_name="s"),
        scratch_types=[
            pltpu.VMEM_SHARED((N_ACC,), jnp.float32),
            pltpu.VMEM((CHUNKS_PER_W, CHUNK), jnp.int32),
            pltpu.VMEM((CHUNK,), jnp.float32),
            pltpu.VMEM((CHUNK,), jnp.float32),
        ],
    )(row2d)


def _edge_body(cr3d, z, out_p, acc, crbuf0, crbuf1, rows0, rows1,
               semg0, semg1, sems0, sems1, semi0):
    c = lax.axis_index("c")
    s = lax.axis_index("s")
    wid = c * NS + s
    base = wid * CHUNKS_PER_W
    def zbody(i, carry):
        for j in range(D // 16):
            rows0[i, pl.ds(j * 16, 16)] = jnp.zeros((16,), jnp.float32)
        return carry

    lax.fori_loop(0, CHUNK, zbody, 0)
    for k in range(N_ACC // NS // CHUNK):
        pltpu.sync_copy(rows0, acc.at[pl.ds(s * (N_ACC // NS) + k * CHUNK, CHUNK)])
    plsc.subcore_barrier()

    pltpu.sync_copy(cr3d.at[base], crbuf0)
    pltpu.sync_copy(cr3d.at[base + 1], crbuf1)
    pltpu.async_copy(z.at[crbuf0.at[0]], rows0, semg0)
    np = CHUNKS_PER_W // 2

    def body(t, carry):
        a = base + 2 * t
        pltpu.async_copy(z.at[crbuf1.at[0]], rows1, semg1)
        pltpu.make_async_copy(z.at[crbuf0.at[0]], rows0, semg0).wait()
        sc0 = pltpu.async_copy(rows0, acc.at[crbuf0.at[1]], sems0, add=True)
        sc0.wait()

        @pl.when(t < np - 1)
        def _():
            pltpu.async_copy(cr3d.at[a + 2], crbuf0, semi0)

        pltpu.make_async_copy(z.at[crbuf1.at[0]], rows1, semg1).wait()
        sc1 = pltpu.async_copy(rows1, acc.at[crbuf1.at[1]], sems1, add=True)

        @pl.when(t < np - 1)
        def _():
            pltpu.make_async_copy(cr3d.at[a + 2], crbuf0, semi0).wait()
            pltpu.async_copy(z.at[crbuf0.at[0]], rows0, semg0)

        sc1.wait()

        @pl.when(t < np - 1)
        def _():
            pltpu.sync_copy(cr3d.at[a + 3], crbuf1)

        return carry

    lax.fori_loop(0, np, body, 0)
    plsc.subcore_barrier()
    rows_per_tile = N_ACC // NS
    pltpu.sync_copy(acc.at[pl.ds(s * rows_per_tile, rows_per_tile)],
                    out_p.at[c, pl.ds(s * rows_per_tile, rows_per_tile)])


def _edge_kernel(cr3d, z):
    return pl.kernel(
        _edge_body,
        out_type=jax.ShapeDtypeStruct((NC, N_ACC, D), jnp.float32),
        mesh=plsc.VectorSubcoreMesh(core_axis_name="c", subcore_axis_name="s"),
        scratch_types=[
            pltpu.VMEM_SHARED((N_ACC, D), jnp.float32),
            pltpu.VMEM((2, CHUNK), jnp.int32),
            pltpu.VMEM((2, CHUNK), jnp.int32),
            pltpu.VMEM((CHUNK, D), jnp.float32),
            pltpu.VMEM((CHUNK, D), jnp.float32),
            pltpu.SemaphoreType.DMA,
            pltpu.SemaphoreType.DMA,
            pltpu.SemaphoreType.DMA,
            pltpu.SemaphoreType.DMA,
            pltpu.SemaphoreType.DMA,
        ],
    )(cr3d, z)


def _dis_from(deg_ref):
    d = deg_ref[..., 0:1] + deg_ref[..., 1:2]
    return jnp.where(d > 0.0, lax.rsqrt(d), 0.0)


def _z_body(x_ref, w_ref, deg_ref, o_ref):
    mm = lax.dot_general(x_ref[...], w_ref[...], (((1,), (1,)), ((), ())),
                         preferred_element_type=jnp.float32)
    o_ref[...] = mm * _dis_from(deg_ref)


def _z_kernel(x, w, deg_t):
    blk = 1000
    grid = N_NODES // blk
    return pl.pallas_call(
        _z_body,
        grid=(grid,),
        in_specs=[
            pl.BlockSpec((blk, D), lambda i: (i, 0)),
            pl.BlockSpec((D, D), lambda i: (0, 0)),
            pl.BlockSpec((blk, NC), lambda i: (i, 0)),
        ],
        out_specs=pl.BlockSpec((blk, D), lambda i: (i, 0)),
        out_shape=jax.ShapeDtypeStruct((N_NODES, D), jnp.float32),
    )(x, w, deg_t)


def _final_body(p_ref, deg_ref, b_ref, o_ref):
    acc = p_ref[0] + p_ref[1]
    o_ref[...] = acc * _dis_from(deg_ref) + b_ref[...]


def _final_kernel(out_p, deg_t, b2):
    blk = 1000
    grid = N_NODES // blk
    return pl.pallas_call(
        _final_body,
        grid=(grid,),
        in_specs=[
            pl.BlockSpec((NC, blk, D), lambda i: (0, i, 0)),
            pl.BlockSpec((blk, NC), lambda i: (i, 0)),
            pl.BlockSpec((1, D), lambda i: (0, 0)),
        ],
        out_specs=pl.BlockSpec((blk, D), lambda i: (i, 0)),
        out_shape=jax.ShapeDtypeStruct((N_NODES, D), jnp.float32),
    )(out_p, deg_t, b2)


@jax.jit
def kernel(x, edge_index, W, b):
    row = edge_index[0]
    col = edge_index[1]
    pad = E_PAD - N_EDGES
    pad_row = N_NODES + (jnp.arange(pad, dtype=jnp.int32) % N_DUMP)
    pad_col = jnp.zeros((pad,), jnp.int32)
    row2d = jnp.concatenate([row, pad_row]).reshape(E_PAD // CHUNK, CHUNK)
    col2d = jnp.concatenate([col, pad_col]).reshape(E_PAD // CHUNK, CHUNK)
    cr3d = jnp.stack([col2d, row2d], axis=1)

    deg_p = _deg_kernel(row2d).reshape(NC, N_ACC)
    deg_t = deg_p[:, :N_NODES].T
    z = _z_kernel(x, W, deg_t)
    out_p = _edge_kernel(cr3d, z)
    return _final_kernel(out_p[:, :N_NODES], deg_t, b.reshape(1, D))


# speedup vs baseline: 12.0762x; 1.0015x over previous
import functools

import jax
import jax.numpy as jnp
from jax import lax
from jax.experimental import pallas as pl
from jax.experimental.pallas import tpu as pltpu
from jax.experimental.pallas import tpu_sc as plsc

N_NODES = 10000
N_EDGES = 320000
D = 128

NC = 2
NS = 16
NW = NC * NS
CHUNK = 128
CHUNKS_PER_W = 80
E_PAD = NW * CHUNK * CHUNKS_PER_W
N_DUMP = 16
N_ACC = 10240


def _deg_body(row2d, deg_out, deg_acc, rstage, ones_v, zeros_v):
    c = lax.axis_index("c")
    s = lax.axis_index("s")
    wid = c * NS + s
    for j in range(CHUNK // 16):
        ones_v[pl.ds(j * 16, 16)] = jnp.ones((16,), jnp.float32)
        zeros_v[pl.ds(j * 16, 16)] = jnp.zeros((16,), jnp.float32)
    for k in range(N_ACC // NS // CHUNK):
        pltpu.sync_copy(zeros_v, deg_acc.at[pl.ds(s * (N_ACC // NS) + k * CHUNK, CHUNK)])
    plsc.subcore_barrier()
    pltpu.sync_copy(row2d.at[pl.ds(wid * CHUNKS_PER_W, CHUNKS_PER_W)], rstage)

    def body(g, carry):
        pltpu.sync_copy(ones_v, deg_acc.at[rstage.at[g]], add=True)
        return carry

    lax.fori_loop(0, CHUNKS_PER_W, body, 0)
    plsc.subcore_barrier()
    pltpu.sync_copy(deg_acc.at[pl.ds(s * (N_ACC // NS), N_ACC // NS)],
                    deg_out.at[pl.ds(c * N_ACC + s * (N_ACC // NS), N_ACC // NS)])


def _deg_kernel(row2d):
    return pl.kernel(
        _deg_body,
        out_type=jax.ShapeDtypeStruct((NC * N_ACC,), jnp.float32),
        mesh=plsc.VectorSubcoreMesh(core_axis_name="c", subcore_axis_name="s"),
        scratch_types=[
            pltpu.VMEM_SHARED((N_ACC,), jnp.float32),
            pltpu.VMEM((CHUNKS_PER_W, CHUNK), jnp.int32),
            pltpu.VMEM((CHUNK,), jnp.float32),
            pltpu.VMEM((CHUNK,), jnp.float32),
        ],
    )(row2d)


def _edge_body(cr3d, z, out_p, acc, crbuf0, crbuf1, rows0, rows1,
               semg0, semg1, sems0, sems1, semi0):
    c = lax.axis_index("c")
    s = lax.axis_index("s")
    wid = c * NS + s
    base = wid * CHUNKS_PER_W
    def zbody(i, carry):
        for j in range(D // 16):
            rows0[i, pl.ds(j * 16, 16)] = jnp.zeros((16,), jnp.float32)
        return carry

    lax.fori_loop(0, CHUNK, zbody, 0)
    for k in range(N_ACC // NS // CHUNK):
        pltpu.sync_copy(rows0, acc.at[pl.ds(s * (N_ACC // NS) + k * CHUNK, CHUNK)])
    plsc.subcore_barrier()

    pltpu.sync_copy(cr3d.at[base], crbuf0)
    pltpu.sync_copy(cr3d.at[base + 1], crbuf1)
    pltpu.async_copy(z.at[crbuf0.at[0]], rows0, semg0)
    np = CHUNKS_PER_W // 2

    def body(t, carry):
        a = base + 2 * t
        pltpu.async_copy(z.at[crbuf1.at[0]], rows1, semg1)
        pltpu.make_async_copy(z.at[crbuf0.at[0]], rows0, semg0).wait()
        sc0 = pltpu.async_copy(rows0, acc.at[crbuf0.at[1]], sems0, add=True)
        sc0.wait()

        @pl.when(t < np - 1)
        def _():
            pltpu.async_copy(cr3d.at[a + 2], crbuf0, semi0)

        pltpu.make_async_copy(z.at[crbuf1.at[0]], rows1, semg1).wait()
        sc1 = pltpu.async_copy(rows1, acc.at[crbuf1.at[1]], sems1, add=True)

        @pl.when(t < np - 1)
        def _():
            pltpu.make_async_copy(cr3d.at[a + 2], crbuf0, semi0).wait()
            pltpu.async_copy(z.at[crbuf0.at[0]], rows0, semg0)

        sc1.wait()

        @pl.when(t < np - 1)
        def _():
            pltpu.sync_copy(cr3d.at[a + 3], crbuf1)

        return carry

    lax.fori_loop(0, np, body, 0)
    plsc.subcore_barrier()
    rows_per_tile = N_ACC // NS
    pltpu.sync_copy(acc.at[pl.ds(s * rows_per_tile, rows_per_tile)],
                    out_p.at[c, pl.ds(s * rows_per_tile, rows_per_tile)])


def _edge_kernel(cr3d, z):
    return pl.kernel(
        _edge_body,
        out_type=jax.ShapeDtypeStruct((NC, N_ACC, D), jnp.float32),
        mesh=plsc.VectorSubcoreMesh(core_axis_name="c", subcore_axis_name="s"),
        compiler_params=pltpu.CompilerParams(use_tc_tiling_on_sc=False),
        scratch_types=[
            pltpu.VMEM_SHARED((N_ACC, D), jnp.float32),
            pltpu.VMEM((2, CHUNK), jnp.int32),
            pltpu.VMEM((2, CHUNK), jnp.int32),
            pltpu.VMEM((CHUNK, D), jnp.float32),
            pltpu.VMEM((CHUNK, D), jnp.float32),
            pltpu.SemaphoreType.DMA,
            pltpu.SemaphoreType.DMA,
            pltpu.SemaphoreType.DMA,
            pltpu.SemaphoreType.DMA,
            pltpu.SemaphoreType.DMA,
        ],
    )(cr3d, z)


def _dis_from(deg_ref):
    d = deg_ref[..., 0:1] + deg_ref[..., 1:2]
    return jnp.where(d > 0.0, lax.rsqrt(d), 0.0)


def _z_body(x_ref, w_ref, deg_ref, o_ref):
    mm = lax.dot_general(x_ref[...], w_ref[...], (((1,), (1,)), ((), ())),
                         preferred_element_type=jnp.float32)
    o_ref[...] = mm * _dis_from(deg_ref)


def _z_kernel(x, w, deg_t):
    blk = 1000
    grid = N_NODES // blk
    return pl.pallas_call(
        _z_body,
        grid=(grid,),
        in_specs=[
            pl.BlockSpec((blk, D), lambda i: (i, 0)),
            pl.BlockSpec((D, D), lambda i: (0, 0)),
            pl.BlockSpec((blk, NC), lambda i: (i, 0)),
        ],
        out_specs=pl.BlockSpec((blk, D), lambda i: (i, 0)),
        out_shape=jax.ShapeDtypeStruct((N_NODES, D), jnp.float32),
    )(x, w, deg_t)


def _final_body(p_ref, deg_ref, b_ref, o_ref):
    acc = p_ref[0] + p_ref[1]
    o_ref[...] = acc * _dis_from(deg_ref) + b_ref[...]


def _final_kernel(out_p, deg_t, b2):
    blk = 1000
    grid = N_NODES // blk
    return pl.pallas_call(
        _final_body,
        grid=(grid,),
        in_specs=[
            pl.BlockSpec((NC, blk, D), lambda i: (0, i, 0)),
            pl.BlockSpec((blk, NC), lambda i: (i, 0)),
            pl.BlockSpec((1, D), lambda i: (0, 0)),
        ],
        out_specs=pl.BlockSpec((blk, D), lambda i: (i, 0)),
        out_shape=jax.ShapeDtypeStruct((N_NODES, D), jnp.float32),
    )(out_p, deg_t, b2)


@jax.jit
def kernel(x, edge_index, W, b):
    row = edge_index[0]
    col = edge_index[1]
    pad = E_PAD - N_EDGES
    pad_row = N_NODES + (jnp.arange(pad, dtype=jnp.int32) % N_DUMP)
    pad_col = jnp.zeros((pad,), jnp.int32)
    row2d = jnp.concatenate([row, pad_row]).reshape(E_PAD // CHUNK, CHUNK)
    col2d = jnp.concatenate([col, pad_col]).reshape(E_PAD // CHUNK, CHUNK)
    cr3d = jnp.stack([col2d, row2d], axis=1)

    deg_p = _deg_kernel(row2d).reshape(NC, N_ACC)
    deg_t = deg_p[:, :N_NODES].T
    z = _z_kernel(x, W, deg_t)
    out_p = _edge_kernel(cr3d, z)
    return _final_kernel(out_p[:, :N_NODES], deg_t, b.reshape(1, D))


# speedup vs baseline: 17.5457x; 1.4529x over previous
import functools

import numpy as np

import jax
import jax.numpy as jnp
from jax import lax
from jax.experimental import pallas as pl
from jax.experimental.pallas import tpu as pltpu
from jax.experimental.pallas import tpu_sc as plsc

N_NODES = 10000
N_EDGES = 320000
D = 128

NC = 2
NS = 16
NW = NC * NS
CHUNK = 128
CHUNKS_PER_W = 80
E_PAD = NW * CHUNK * CHUNKS_PER_W
N_DUMP = 16
N_ACC = 10240


def _deg_body(row2d, deg_out, deg_acc, rstage, ones_v, zeros_v):
    c = lax.axis_index("c")
    s = lax.axis_index("s")
    wid = c * NS + s
    for j in range(CHUNK // 16):
        ones_v[pl.ds(j * 16, 16)] = jnp.ones((16,), jnp.float32)
        zeros_v[pl.ds(j * 16, 16)] = jnp.zeros((16,), jnp.float32)
    for k in range(N_ACC // NS // CHUNK):
        pltpu.sync_copy(zeros_v, deg_acc.at[pl.ds(s * (N_ACC // NS) + k * CHUNK, CHUNK)])
    plsc.subcore_barrier()
    pltpu.sync_copy(row2d.at[pl.ds(wid * CHUNKS_PER_W, CHUNKS_PER_W)], rstage)

    def body(g, carry):
        pltpu.sync_copy(ones_v, deg_acc.at[rstage.at[g]], add=True)
        return carry

    lax.fori_loop(0, CHUNKS_PER_W, body, 0)
    plsc.subcore_barrier()
    pltpu.sync_copy(deg_acc.at[pl.ds(s * (N_ACC // NS), N_ACC // NS)],
                    deg_out.at[pl.ds(c * N_ACC + s * (N_ACC // NS), N_ACC // NS)])


def _deg_kernel(row2d):
    return pl.kernel(
        _deg_body,
        out_type=jax.ShapeDtypeStruct((NC * N_ACC,), jnp.float32),
        mesh=plsc.VectorSubcoreMesh(core_axis_name="c", subcore_axis_name="s"),
        scratch_types=[
            pltpu.VMEM_SHARED((N_ACC,), jnp.float32),
            pltpu.VMEM((CHUNKS_PER_W, CHUNK), jnp.int32),
            pltpu.VMEM((CHUNK,), jnp.float32),
            pltpu.VMEM((CHUNK,), jnp.float32),
        ],
    )(row2d)


def _convert_rows(gb, fb):
    def cbody(i, carry):
        for k in range(D // 32):
            w = plsc.bitcast(gb[i, pl.ds(32 * k, 32)], jnp.int32)
            lo = plsc.bitcast(lax.shift_left(w, 16), jnp.float32)
            hi = plsc.bitcast(
                lax.bitwise_and(w, jnp.int32(-65536)), jnp.float32)
            fb[i, pl.ds(32 * k, 16)] = lo
            fb[i, pl.ds(32 * k + 16, 16)] = hi
        return carry

    lax.fori_loop(0, CHUNK, cbody, 0)


def _edge_body(cr3d, z, out_p, acc, crbuf0, crbuf1, gb0, gb1, fb,
               semg0, semg1, sems):
    c = lax.axis_index("c")
    s = lax.axis_index("s")
    wid = c * NS + s
    base = wid * CHUNKS_PER_W
    def zbody(i, carry):
        for j in range(D // 16):
            fb[i, pl.ds(j * 16, 16)] = jnp.zeros((16,), jnp.float32)
        return carry

    lax.fori_loop(0, CHUNK, zbody, 0)
    for k in range(N_ACC // NS // CHUNK):
        pltpu.sync_copy(fb, acc.at[pl.ds(s * (N_ACC // NS) + k * CHUNK, CHUNK)])
    plsc.subcore_barrier()

    pltpu.sync_copy(cr3d.at[base], crbuf0)
    pltpu.sync_copy(cr3d.at[base + 1], crbuf1)
    pltpu.async_copy(z.at[crbuf0.at[0]], gb0, semg0)
    pltpu.async_copy(z.at[crbuf1.at[0]], gb1, semg1)
    np = CHUNKS_PER_W // 2

    def body(t, carry):
        a = base + 2 * t
        pltpu.make_async_copy(z.at[crbuf0.at[0]], gb0, semg0).wait()
        _convert_rows(gb0, fb)
        pltpu.async_copy(fb, acc.at[crbuf0.at[1]], sems, add=True).wait()

        @pl.when(t < np - 1)
        def _():
            pltpu.sync_copy(cr3d.at[a + 2], crbuf0)
            pltpu.async_copy(z.at[crbuf0.at[0]], gb0, semg0)

        pltpu.make_async_copy(z.at[crbuf1.at[0]], gb1, semg1).wait()
        _convert_rows(gb1, fb)
        pltpu.async_copy(fb, acc.at[crbuf1.at[1]], sems, add=True).wait()

        @pl.when(t < np - 1)
        def _():
            pltpu.sync_copy(cr3d.at[a + 3], crbuf1)
            pltpu.async_copy(z.at[crbuf1.at[0]], gb1, semg1)

        return carry

    lax.fori_loop(0, np, body, 0)
    pltpu.sync_copy(acc.at[pl.ds(s * (N_ACC // NS), 8)], fb.at[pl.ds(0, 8)])
    plsc.subcore_barrier()
    plsc.subcore_barrier()
    rows_per_tile = N_ACC // NS
    pltpu.sync_copy(acc.at[pl.ds(s * rows_per_tile, rows_per_tile)],
                    out_p.at[c, pl.ds(s * rows_per_tile, rows_per_tile)])


def _edge_kernel(cr3d, z):
    return pl.kernel(
        _edge_body,
        out_type=jax.ShapeDtypeStruct((NC, N_ACC, D), jnp.float32),
        mesh=plsc.VectorSubcoreMesh(core_axis_name="c", subcore_axis_name="s"),
        compiler_params=pltpu.CompilerParams(use_tc_tiling_on_sc=False,
                                             needs_layout_passes=False),
        scratch_types=[
            pltpu.VMEM_SHARED((N_ACC, D), jnp.float32),
            pltpu.VMEM((2, CHUNK), jnp.int32),
            pltpu.VMEM((2, CHUNK), jnp.int32),
            pltpu.VMEM((CHUNK, D), jnp.bfloat16),
            pltpu.VMEM((CHUNK, D), jnp.bfloat16),
            pltpu.VMEM((CHUNK, D), jnp.float32),
            pltpu.SemaphoreType.DMA,
            pltpu.SemaphoreType.DMA,
            pltpu.SemaphoreType.DMA,
        ],
    )(cr3d, z)


def _dis_from(deg_ref):
    d = deg_ref[..., 0:1] + deg_ref[..., 1:2]
    return jnp.where(d > 0.0, lax.rsqrt(d), 0.0)


def _z_body(x_ref, w_ref, deg_ref, o_ref):
    mm = lax.dot_general(x_ref[...], w_ref[...], (((1,), (1,)), ((), ())),
                         preferred_element_type=jnp.float32)
    o_ref[...] = (mm * _dis_from(deg_ref)).astype(jnp.bfloat16)


def _z_kernel(x, w, deg_t):
    blk = 1000
    grid = N_NODES // blk
    return pl.pallas_call(
        _z_body,
        grid=(grid,),
        in_specs=[
            pl.BlockSpec((blk, D), lambda i: (i, 0)),
            pl.BlockSpec((D, D), lambda i: (0, 0)),
            pl.BlockSpec((blk, NC), lambda i: (i, 0)),
        ],
        out_specs=pl.BlockSpec((blk, D), lambda i: (i, 0)),
        out_shape=jax.ShapeDtypeStruct((N_NODES, D), jnp.bfloat16),
    )(x, w, deg_t)


def _final_body(p_ref, deg_ref, b_ref, o_ref):
    acc = p_ref[0] + p_ref[1]
    o_ref[...] = acc * _dis_from(deg_ref) + b_ref[...]


def _final_kernel(out_p, deg_t, b2):
    blk = 1000
    grid = N_NODES // blk
    return pl.pallas_call(
        _final_body,
        grid=(grid,),
        in_specs=[
            pl.BlockSpec((NC, blk, D), lambda i: (0, i, 0)),
            pl.BlockSpec((blk, NC), lambda i: (i, 0)),
            pl.BlockSpec((1, D), lambda i: (0, 0)),
        ],
        out_specs=pl.BlockSpec((blk, D), lambda i: (i, 0)),
        out_shape=jax.ShapeDtypeStruct((N_NODES, D), jnp.float32),
    )(out_p, deg_t, b2)


@jax.jit
def kernel(x, edge_index, W, b):
    row = edge_index[0]
    col = edge_index[1]
    pad = E_PAD - N_EDGES
    pad_row = N_NODES + (jnp.arange(pad, dtype=jnp.int32) % N_DUMP)
    pad_col = jnp.zeros((pad,), jnp.int32)
    row2d = jnp.concatenate([row, pad_row]).reshape(E_PAD // CHUNK, CHUNK)
    col2d = jnp.concatenate([col, pad_col]).reshape(E_PAD // CHUNK, CHUNK)
    cr3d = jnp.stack([col2d, row2d], axis=1)

    perm = np.empty((D,), dtype=np.int64)
    for k in range(D // 32):
        for j in range(16):
            perm[32 * k + j] = 32 * k + 2 * j
            perm[32 * k + 16 + j] = 32 * k + 2 * j + 1
    inv_perm = np.argsort(perm)
    W2 = W[jnp.asarray(inv_perm)]

    deg_p = _deg_kernel(row2d).reshape(NC, N_ACC)
    deg_t = deg_p[:, :N_NODES].T
    z = _z_kernel(x, W2, deg_t)
    out_p = _edge_kernel(cr3d, z)
    return _final_kernel(out_p[:, :N_NODES], deg_t, b.reshape(1, D))


# speedup vs baseline: 18.0288x; 1.0275x over previous
import functools

import numpy as np

import jax
import jax.numpy as jnp
from jax import lax
from jax.experimental import pallas as pl
from jax.experimental.pallas import tpu as pltpu
from jax.experimental.pallas import tpu_sc as plsc

N_NODES = 10000
N_EDGES = 320000
D = 128

NC = 2
NS = 16
NW = NC * NS
CHUNK = 128
CHUNKS_PER_W = 80
E_PAD = NW * CHUNK * CHUNKS_PER_W
N_DUMP = 16
N_ACC = 10240


def _deg_body(row2d, deg_out, deg_acc, rstage, ones_v, zeros_v,
              semd0, semd1, semd2, semd3):
    c = lax.axis_index("c")
    s = lax.axis_index("s")
    wid = c * NS + s
    for j in range(CHUNK // 16):
        ones_v[pl.ds(j * 16, 16)] = jnp.ones((16,), jnp.float32)
        zeros_v[pl.ds(j * 16, 16)] = jnp.zeros((16,), jnp.float32)
    for k in range(N_ACC // NS // CHUNK):
        pltpu.sync_copy(zeros_v, deg_acc.at[pl.ds(s * (N_ACC // NS) + k * CHUNK, CHUNK)])
    plsc.subcore_barrier()
    pltpu.sync_copy(row2d.at[pl.ds(wid * CHUNKS_PER_W, CHUNKS_PER_W)], rstage)

    sems = (semd0, semd1, semd2, semd3)
    for b in range(4):
        pltpu.async_copy(ones_v, deg_acc.at[rstage.at[b]], sems[b], add=True)

    def body(t, carry):
        for b in range(4):
            g = 4 * t + b
            pltpu.make_async_copy(ones_v, deg_acc.at[rstage.at[g]], sems[b]).wait()

            @pl.when(t < CHUNKS_PER_W // 4 - 1)
            def _():
                pltpu.async_copy(ones_v, deg_acc.at[rstage.at[g + 4]], sems[b],
                                 add=True)

        return carry

    lax.fori_loop(0, CHUNKS_PER_W // 4, body, 0)
    pltpu.sync_copy(deg_acc.at[pl.ds(s * (N_ACC // NS), CHUNK)], zeros_v)
    plsc.subcore_barrier()
    plsc.subcore_barrier()
    pltpu.sync_copy(deg_acc.at[pl.ds(s * (N_ACC // NS), N_ACC // NS)],
                    deg_out.at[pl.ds(c * N_ACC + s * (N_ACC // NS), N_ACC // NS)])


def _deg_kernel(row2d):
    return pl.kernel(
        _deg_body,
        out_type=jax.ShapeDtypeStruct((NC * N_ACC,), jnp.float32),
        mesh=plsc.VectorSubcoreMesh(core_axis_name="c", subcore_axis_name="s"),
        scratch_types=[
            pltpu.VMEM_SHARED((N_ACC,), jnp.float32),
            pltpu.VMEM((CHUNKS_PER_W, CHUNK), jnp.int32),
            pltpu.VMEM((CHUNK,), jnp.float32),
            pltpu.VMEM((CHUNK,), jnp.float32),
            pltpu.SemaphoreType.DMA,
            pltpu.SemaphoreType.DMA,
            pltpu.SemaphoreType.DMA,
            pltpu.SemaphoreType.DMA,
        ],
    )(row2d)


def _convert_rows(gb, fb):
    def cbody(i, carry):
        for k in range(D // 32):
            w = plsc.bitcast(gb[i, pl.ds(32 * k, 32)], jnp.int32)
            lo = plsc.bitcast(lax.shift_left(w, 16), jnp.float32)
            hi = plsc.bitcast(
                lax.bitwise_and(w, jnp.int32(-65536)), jnp.float32)
            fb[i, pl.ds(32 * k, 16)] = lo
            fb[i, pl.ds(32 * k + 16, 16)] = hi
        return carry

    lax.fori_loop(0, CHUNK, cbody, 0)


def _edge_body(cr3d, z, out_p, acc, crbuf0, crbuf1, gb0, gb1, fb,
               semg0, semg1, sems):
    c = lax.axis_index("c")
    s = lax.axis_index("s")
    wid = c * NS + s
    base = wid * CHUNKS_PER_W
    def zbody(i, carry):
        for j in range(D // 16):
            fb[i, pl.ds(j * 16, 16)] = jnp.zeros((16,), jnp.float32)
        return carry

    lax.fori_loop(0, CHUNK, zbody, 0)
    for k in range(N_ACC // NS // CHUNK):
        pltpu.sync_copy(fb, acc.at[pl.ds(s * (N_ACC // NS) + k * CHUNK, CHUNK)])
    plsc.subcore_barrier()

    pltpu.sync_copy(cr3d.at[base], crbuf0)
    pltpu.sync_copy(cr3d.at[base + 1], crbuf1)
    pltpu.async_copy(z.at[crbuf0.at[0]], gb0, semg0)
    pltpu.async_copy(z.at[crbuf1.at[0]], gb1, semg1)
    np = CHUNKS_PER_W // 2

    def body(t, carry):
        a = base + 2 * t
        pltpu.make_async_copy(z.at[crbuf0.at[0]], gb0, semg0).wait()
        _convert_rows(gb0, fb)
        pltpu.async_copy(fb, acc.at[crbuf0.at[1]], sems, add=True).wait()

        @pl.when(t < np - 1)
        def _():
            pltpu.sync_copy(cr3d.at[a + 2], crbuf0)
            pltpu.async_copy(z.at[crbuf0.at[0]], gb0, semg0)

        pltpu.make_async_copy(z.at[crbuf1.at[0]], gb1, semg1).wait()
        _convert_rows(gb1, fb)
        pltpu.async_copy(fb, acc.at[crbuf1.at[1]], sems, add=True).wait()

        @pl.when(t < np - 1)
        def _():
            pltpu.sync_copy(cr3d.at[a + 3], crbuf1)
            pltpu.async_copy(z.at[crbuf1.at[0]], gb1, semg1)

        return carry

    lax.fori_loop(0, np, body, 0)
    pltpu.sync_copy(acc.at[pl.ds(s * (N_ACC // NS), 8)], fb.at[pl.ds(0, 8)])
    plsc.subcore_barrier()
    plsc.subcore_barrier()
    rows_per_tile = N_ACC // NS
    pltpu.sync_copy(acc.at[pl.ds(s * rows_per_tile, rows_per_tile)],
                    out_p.at[c, pl.ds(s * rows_per_tile, rows_per_tile)])


def _edge_kernel(cr3d, z):
    return pl.kernel(
        _edge_body,
        out_type=jax.ShapeDtypeStruct((NC, N_ACC, D), jnp.float32),
        mesh=plsc.VectorSubcoreMesh(core_axis_name="c", subcore_axis_name="s"),
        compiler_params=pltpu.CompilerParams(use_tc_tiling_on_sc=False,
                                             needs_layout_passes=False),
        scratch_types=[
            pltpu.VMEM_SHARED((N_ACC, D), jnp.float32),
            pltpu.VMEM((2, CHUNK), jnp.int32),
            pltpu.VMEM((2, CHUNK), jnp.int32),
            pltpu.VMEM((CHUNK, D), jnp.bfloat16),
            pltpu.VMEM((CHUNK, D), jnp.bfloat16),
            pltpu.VMEM((CHUNK, D), jnp.float32),
            pltpu.SemaphoreType.DMA,
            pltpu.SemaphoreType.DMA,
            pltpu.SemaphoreType.DMA,
        ],
    )(cr3d, z)


def _dis_from(deg_ref):
    d = deg_ref[..., 0:1] + deg_ref[..., 1:2]
    return jnp.where(d > 0.0, lax.rsqrt(d), 0.0)


def _z_body(x_ref, w_ref, deg_ref, o_ref):
    mm = lax.dot_general(x_ref[...], w_ref[...], (((1,), (1,)), ((), ())),
                         preferred_element_type=jnp.float32)
    o_ref[...] = (mm * _dis_from(deg_ref)).astype(jnp.bfloat16)


def _z_kernel(x, w, deg_t):
    blk = 1000
    grid = N_NODES // blk
    return pl.pallas_call(
        _z_body,
        grid=(grid,),
        in_specs=[
            pl.BlockSpec((blk, D), lambda i: (i, 0)),
            pl.BlockSpec((D, D), lambda i: (0, 0)),
            pl.BlockSpec((blk, NC), lambda i: (i, 0)),
        ],
        out_specs=pl.BlockSpec((blk, D), lambda i: (i, 0)),
        out_shape=jax.ShapeDtypeStruct((N_NODES, D), jnp.bfloat16),
    )(x, w, deg_t)


def _final_body(p_ref, deg_ref, b_ref, o_ref):
    acc = p_ref[0] + p_ref[1]
    o_ref[...] = acc * _dis_from(deg_ref) + b_ref[...]


def _final_kernel(out_p, deg_t, b2):
    blk = 1000
    grid = N_NODES // blk
    return pl.pallas_call(
        _final_body,
        grid=(grid,),
        in_specs=[
            pl.BlockSpec((NC, blk, D), lambda i: (0, i, 0)),
            pl.BlockSpec((blk, NC), lambda i: (i, 0)),
            pl.BlockSpec((1, D), lambda i: (0, 0)),
        ],
        out_specs=pl.BlockSpec((blk, D), lambda i: (i, 0)),
        out_shape=jax.ShapeDtypeStruct((N_NODES, D), jnp.float32),
    )(out_p, deg_t, b2)


@jax.jit
def kernel(x, edge_index, W, b):
    row = edge_index[0]
    col = edge_index[1]
    pad = E_PAD - N_EDGES
    pad_row = N_NODES + (jnp.arange(pad, dtype=jnp.int32) % N_DUMP)
    pad_col = jnp.zeros((pad,), jnp.int32)
    row2d = jnp.concatenate([row, pad_row]).reshape(E_PAD // CHUNK, CHUNK)
    col2d = jnp.concatenate([col, pad_col]).reshape(E_PAD // CHUNK, CHUNK)
    cr3d = jnp.stack([col2d, row2d], axis=1)

    perm = np.empty((D,), dtype=np.int64)
    for k in range(D // 32):
        for j in range(16):
            perm[32 * k + j] = 32 * k + 2 * j
            perm[32 * k + 16 + j] = 32 * k + 2 * j + 1
    inv_perm = np.argsort(perm)
    W2 = W[jnp.asarray(inv_perm)]

    deg_p = _deg_kernel(row2d).reshape(NC, N_ACC)
    deg_t = deg_p[:, :N_NODES].T
    z = _z_kernel(x, W2, deg_t)
    out_p = _edge_kernel(cr3d, z)
    return _final_kernel(out_p, deg_t, b.reshape(1, D))
